# baseline (speedup 1.0000x reference)
import functools

import jax
import jax.numpy as jnp
from jax import lax
from jax.experimental import pallas as pl
from jax.experimental.pallas import tpu as pltpu
from jax.experimental.pallas import tpu_sc as plsc

N = 10000
E = 320000
D = 128

NC = 2
NS = 16
NW = NC * NS
EPW = E // NW
C = 128
CHUNKS = -(-EPW // C)
EPAD = CHUNKS * C
NB = 3
ZBLK = 80
NZB = N // ZBLK
OBLK = 200
NOB = N // OBLK


def _sc_spmm(edr_hbm, edc_hbm, emb_hbm, part_hbm, acc,
             msg, ebr, ebc, gsem, rsem, csem, ssem):
    cid = lax.axis_index("c")
    sid = lax.axis_index("s")
    wid = cid * NS + sid

    def zero_row(i, _):
        for d in range(D // 16):
            msg[0][i, pl.ds(d * 16, 16)] = jnp.zeros((16,), jnp.float32)
        return 0

    lax.fori_loop(0, ZBLK, zero_row, 0)

    def zero_blk(k, _):
        b = sid + k * NS

        @pl.when(b < NZB)
        def _():
            pltpu.sync_copy(msg[0].at[pl.ds(0, ZBLK)],
                            acc.at[pl.ds(b * ZBLK, ZBLK)])

        return 0

    lax.fori_loop(0, -(-NZB // NS), zero_blk, 0)
    plsc.subcore_barrier()

    def start_edr(ci, b):
        pltpu.async_copy(edr_hbm.at[wid, ci], ebr[b], rsem[b])

    def wait_edr(b):
        pltpu.make_async_copy(edr_hbm.at[0, 0], ebr[b], rsem[b]).wait()

    def start_edc(ci, b):
        pltpu.async_copy(edc_hbm.at[wid, ci], ebc[b], csem[b])

    def wait_edc(b):
        pltpu.make_async_copy(edc_hbm.at[0, 0], ebc[b], csem[b]).wait()

    def start_gather(ci, b):
        pltpu.async_copy(emb_hbm.at[ebc[b].at[0]], msg[b], gsem[b])

    def wait_gather(b):
        pltpu.make_async_copy(emb_hbm.at[pl.ds(0, C)], msg[b], gsem[b]).wait()

    def start_scatter(ci, b):
        pltpu.async_copy(msg[b], acc.at[ebr[b].at[0]], ssem[b], add=True)

    def wait_scatter(b):
        pltpu.make_async_copy(emb_hbm.at[pl.ds(0, C)], msg[b], ssem[b]).wait()

    def scale(b):
        def grp(g, _):
            bits = ebc[b][1, pl.ds(g * 16, 16)]
            gvv = lax.bitcast_convert_type(bits, jnp.float32) * 0.75
            for j in range(16):
                gvb = jnp.full((16,), gvv[j], jnp.float32)
                i = g * 16 + j
                for d in range(D // 16):
                    sl = pl.ds(d * 16, 16)
                    msg[b][i, sl] = msg[b][i, sl] * gvb
            return 0

        lax.fori_loop(0, C // 16, grp, 0)

    def step(ci, b):
        @pl.when(ci >= 2)
        def _():
            wait_scatter((b + 1) % NB)

        @pl.when(ci + 1 < CHUNKS)
        def _():
            start_edr(ci + 1, (b + 1) % NB)

        @pl.when(ci + 2 < CHUNKS)
        def _():
            start_edc(ci + 2, (b + 2) % NB)

        @pl.when(ci + 1 < CHUNKS)
        def _():
            wait_edc((b + 1) % NB)
            start_gather(ci + 1, (b + 1) % NB)

        wait_gather(b)
        scale(b)
        wait_edr(b)
        start_scatter(ci, b)

    start_edc(0, 0)
    start_edc(1, 1)
    start_edr(0, 0)
    wait_edc(0)
    start_gather(0, 0)

    def tri(k, _):
        for j in range(NB):
            step(NB * k + j, j)
        return 0

    lax.fori_loop(0, (CHUNKS - 1) // NB, tri, 0)
    step(jnp.int32(CHUNKS - 1), (CHUNKS - 1) % NB)
    wait_scatter(2)
    wait_scatter(0)
    plsc.subcore_barrier()

    def out_body(k, _):
        b = sid + k * NS

        @pl.when(b < NOB)
        def _():
            row0 = b * OBLK
            pltpu.sync_copy(acc.at[pl.ds(row0, OBLK)],
                            part_hbm.at[cid, pl.ds(row0, OBLK)])

        return 0

    lax.fori_loop(0, -(-NOB // NS), out_body, 0)


def _combine_body(p_ref, e_ref, o_ref):
    o_ref[...] = p_ref[0] + p_ref[1] + 0.25 * e_ref[...]


def kernel(g_indices, g_values, emb_weight):
    pad = EPAD - EPW
    rows = g_indices[0].astype(jnp.int32).reshape(NW, EPW)
    cols = g_indices[1].astype(jnp.int32).reshape(NW, EPW)
    gvb = jax.lax.bitcast_convert_type(
        g_values.astype(jnp.float32), jnp.int32).reshape(NW, EPW)
    zp = jnp.zeros((NW, pad), jnp.int32)
    rows = jnp.concatenate([rows, zp], axis=1).reshape(NW, CHUNKS, 1, C)
    cols = jnp.concatenate([cols, zp], axis=1).reshape(NW, CHUNKS, C)
    gvb = jnp.concatenate([gvb, zp], axis=1).reshape(NW, CHUNKS, C)
    edc = jnp.stack([cols, gvb], axis=2)
    emb = emb_weight.astype(jnp.float32)

    mesh = plsc.VectorSubcoreMesh(core_axis_name="c", subcore_axis_name="s")
    spmm = functools.partial(
        pl.kernel,
        out_type=jax.ShapeDtypeStruct((NC, N, D), jnp.float32),
        mesh=mesh,
        scratch_types=[
            pltpu.VMEM_SHARED((N, D), jnp.float32),
            [pltpu.VMEM((C, D), jnp.float32) for _ in range(NB)],
            [pltpu.VMEM((1, C), jnp.int32) for _ in range(NB)],
            [pltpu.VMEM((2, C), jnp.int32) for _ in range(NB)],
            [pltpu.SemaphoreType.DMA for _ in range(NB)],
            [pltpu.SemaphoreType.DMA for _ in range(NB)],
            [pltpu.SemaphoreType.DMA for _ in range(NB)],
            [pltpu.SemaphoreType.DMA for _ in range(NB)],
        ],
    )(_sc_spmm)
    partials = spmm(rows, edc, emb)

    blk = 1000
    out = pl.pallas_call(
        _combine_body,
        grid=(N // blk,),
        in_specs=[
            pl.BlockSpec((NC, blk, D), lambda i: (0, i, 0)),
            pl.BlockSpec((blk, D), lambda i: (i, 0)),
        ],
        out_specs=pl.BlockSpec((blk, D), lambda i: (i, 0)),
        out_shape=jax.ShapeDtypeStruct((N, D), jnp.float32),
    )(partials, emb)
    return out


# speedup vs baseline: 1.6752x; 1.6752x over previous
import functools

import jax
import jax.numpy as jnp
from jax import lax
from jax.experimental import pallas as pl
from jax.experimental.pallas import tpu as pltpu
from jax.experimental.pallas import tpu_sc as plsc

N = 10000
E = 320000
D = 128

NC = 2
NS = 16
NW = NC * NS
EPW = E // NW
C = 80
CHUNKS = EPW // C
NB = 4
OBLK = 200
NOB = N // OBLK


def _sc_spmm(edr_hbm, edc_hbm, emb_hbm, part_hbm, acc,
             msg, ebr, ebc, gsem, rsem, csem, ssem):
    cid = lax.axis_index("c")
    sid = lax.axis_index("s")
    wid = cid * NS + sid

    def zero_row(i, _):
        for d in range(D // 16):
            msg[0][i, pl.ds(d * 16, 16)] = jnp.zeros((16,), jnp.float32)
        return 0

    lax.fori_loop(0, C, zero_row, 0)

    def zero_blk(k, _):
        b = sid + k * NS

        @pl.when(b < CHUNKS)
        def _():
            pltpu.sync_copy(msg[0], acc.at[pl.ds(b * C, C)])

        return 0

    lax.fori_loop(0, -(-CHUNKS // NS), zero_blk, 0)
    plsc.subcore_barrier()

    def start_edr(ci, b):
        pltpu.async_copy(edr_hbm.at[wid, ci], ebr[b], rsem[b])

    def wait_edr(b):
        pltpu.make_async_copy(edr_hbm.at[0, 0], ebr[b], rsem[b]).wait()

    def start_edc(ci, b):
        pltpu.async_copy(edc_hbm.at[wid, ci], ebc[b], csem[b])

    def wait_edc(b):
        pltpu.make_async_copy(edc_hbm.at[0, 0], ebc[b], csem[b]).wait()

    def start_gather(ci, b):
        pltpu.async_copy(emb_hbm.at[ebc[b].at[0]], msg[b], gsem[b])

    def wait_gather(b):
        pltpu.make_async_copy(emb_hbm.at[pl.ds(0, C)], msg[b], gsem[b]).wait()

    def start_scatter(ci, b):
        pltpu.async_copy(msg[b], acc.at[ebr[b].at[0]], ssem[b], add=True)

    def wait_scatter(b):
        pltpu.make_async_copy(emb_hbm.at[pl.ds(0, C)], msg[b], ssem[b]).wait()

    def scale(b):
        def grp(g, _):
            bits = ebc[b][1, pl.ds(g * 16, 16)]
            gvv = lax.bitcast_convert_type(bits, jnp.float32) * 0.75
            for j in range(16):
                gvb = jnp.full((16,), gvv[j], jnp.float32)
                i = g * 16 + j
                for d in range(D // 16):
                    sl = pl.ds(d * 16, 16)
                    msg[b][i, sl] = msg[b][i, sl] * gvb
            return 0

        lax.fori_loop(0, C // 16, grp, 0)

    def step(ci, b):
        @pl.when(ci >= 2)
        def _():
            wait_scatter((b + 2) % NB)

        @pl.when(ci + 2 < CHUNKS)
        def _():
            start_edr(ci + 2, (b + 2) % NB)

        @pl.when(ci + 3 < CHUNKS)
        def _():
            start_edc(ci + 3, (b + 3) % NB)

        @pl.when(ci + 2 < CHUNKS)
        def _():
            wait_edc((b + 2) % NB)
            start_gather(ci + 2, (b + 2) % NB)

        wait_gather(b)
        scale(b)
        wait_edr(b)
        start_scatter(ci, b)

    start_edc(0, 0)
    start_edc(1, 1)
    start_edc(2, 2)
    start_edr(0, 0)
    start_edr(1, 1)
    wait_edc(0)
    start_gather(0, 0)
    wait_edc(1)
    start_gather(1, 1)

    def quad(k, _):
        for j in range(NB):
            step(NB * k + j, j)
        return 0

    lax.fori_loop(0, (CHUNKS - 1) // NB, quad, 0)
    wait_scatter(2)
    wait_gather(0)
    scale(0)
    wait_edr(0)
    start_scatter(CHUNKS - 1, 0)
    wait_scatter(3)
    wait_scatter(0)
    plsc.subcore_barrier()

    def out_body(k, _):
        b = sid + k * NS

        @pl.when(b < NOB)
        def _():
            row0 = b * OBLK
            pltpu.sync_copy(acc.at[pl.ds(row0, OBLK)],
                            part_hbm.at[cid, pl.ds(row0, OBLK)])

        return 0

    lax.fori_loop(0, -(-NOB // NS), out_body, 0)


def _combine_body(p_ref, e_ref, o_ref):
    o_ref[...] = p_ref[0] + p_ref[1] + 0.25 * e_ref[...]


def kernel(g_indices, g_values, emb_weight):
    rows = g_indices[0].astype(jnp.int32).reshape(NW, CHUNKS, 1, C)
    cols = g_indices[1].astype(jnp.int32).reshape(NW, CHUNKS, C)
    gvb = jax.lax.bitcast_convert_type(
        g_values.astype(jnp.float32), jnp.int32).reshape(NW, CHUNKS, C)
    edc = jnp.stack([cols, gvb], axis=2)
    emb = emb_weight.astype(jnp.float32)

    mesh = plsc.VectorSubcoreMesh(core_axis_name="c", subcore_axis_name="s")
    spmm = functools.partial(
        pl.kernel,
        out_type=jax.ShapeDtypeStruct((NC, N, D), jnp.float32),
        mesh=mesh,
        scratch_types=[
            pltpu.VMEM_SHARED((N, D), jnp.float32),
            [pltpu.VMEM((C, D), jnp.float32) for _ in range(NB)],
            [pltpu.VMEM((1, C), jnp.int32) for _ in range(NB)],
            [pltpu.VMEM((2, C), jnp.int32) for _ in range(NB)],
            [pltpu.SemaphoreType.DMA for _ in range(NB)],
            [pltpu.SemaphoreType.DMA for _ in range(NB)],
            [pltpu.SemaphoreType.DMA for _ in range(NB)],
            [pltpu.SemaphoreType.DMA for _ in range(NB)],
        ],
    )(_sc_spmm)
    partials = spmm(rows, edc, emb)

    blk = 1000
    out = pl.pallas_call(
        _combine_body,
        grid=(N // blk,),
        in_specs=[
            pl.BlockSpec((NC, blk, D), lambda i: (0, i, 0)),
            pl.BlockSpec((blk, D), lambda i: (i, 0)),
        ],
        out_specs=pl.BlockSpec((blk, D), lambda i: (i, 0)),
        out_shape=jax.ShapeDtypeStruct((N, D), jnp.float32),
    )(partials, emb)
    return out


# speedup vs baseline: 1.6962x; 1.0125x over previous
import functools

import jax
import jax.numpy as jnp
from jax import lax
from jax.experimental import pallas as pl
from jax.experimental.pallas import tpu as pltpu
from jax.experimental.pallas import tpu_sc as plsc

N = 10000
E = 320000
D = 128

NC = 2
NS = 16
NW = NC * NS
EPW = E // NW
C = 80
CHUNKS = EPW // C
NB = 4
OBLK = 200
NOB = N // OBLK


def _sc_spmm(edr_hbm, edc_hbm, emb_hbm, part_hbm, acc,
             msg, ebr, ebc, gsem, rsem, csem, ssem, zsem):
    cid = lax.axis_index("c")
    sid = lax.axis_index("s")
    wid = cid * NS + sid

    def start_edr(ci, b):
        pltpu.async_copy(edr_hbm.at[wid, ci], ebr[b], rsem[b])

    def wait_edr(b):
        pltpu.make_async_copy(edr_hbm.at[0, 0], ebr[b], rsem[b]).wait()

    def start_edc(ci, b):
        pltpu.async_copy(edc_hbm.at[wid, ci], ebc[b], csem[b])

    def wait_edc(b):
        pltpu.make_async_copy(edc_hbm.at[0, 0], ebc[b], csem[b]).wait()

    def start_gather(ci, b):
        pltpu.async_copy(emb_hbm.at[ebc[b].at[0]], msg[b], gsem[b])

    def wait_gather(b):
        pltpu.make_async_copy(emb_hbm.at[pl.ds(0, C)], msg[b], gsem[b]).wait()

    def start_scatter(ci, b):
        pltpu.async_copy(msg[b], acc.at[ebr[b].at[0]], ssem[b], add=True)

    def wait_scatter(b):
        pltpu.make_async_copy(emb_hbm.at[pl.ds(0, C)], msg[b], ssem[b]).wait()

    def scale(b):
        def grp(g, _):
            bits = ebc[b][1, pl.ds(g * 16, 16)]
            gvv = lax.bitcast_convert_type(bits, jnp.float32)
            for j in range(16):
                gvb = jnp.full((16,), gvv[j], jnp.float32)
                i = g * 16 + j
                for d in range(D // 16):
                    sl = pl.ds(d * 16, 16)
                    msg[b][i, sl] = msg[b][i, sl] * gvb
            return 0

        lax.fori_loop(0, C // 16, grp, 0)

    def step(ci, b):
        @pl.when(ci >= 2)
        def _():
            wait_scatter((b + 2) % NB)

        @pl.when(ci + 2 < CHUNKS)
        def _():
            start_edr(ci + 2, (b + 2) % NB)

        @pl.when(ci + 3 < CHUNKS)
        def _():
            start_edc(ci + 3, (b + 3) % NB)

        @pl.when(ci + 2 < CHUNKS)
        def _():
            wait_edc((b + 2) % NB)
            start_gather(ci + 2, (b + 2) % NB)

        wait_gather(b)
        scale(b)
        wait_edr(b)
        start_scatter(ci, b)

    start_edc(0, 0)
    start_edc(1, 1)
    start_edc(2, 2)
    start_edr(0, 0)
    start_edr(1, 1)
    wait_edc(0)
    start_gather(0, 0)
    wait_edc(1)
    start_gather(1, 1)

    def zero_row(i, _):
        for d in range(D // 16):
            msg[3][i, pl.ds(d * 16, 16)] = jnp.zeros((16,), jnp.float32)
        return 0

    lax.fori_loop(0, C, zero_row, 0)

    def zero_blk(k, _):
        b = sid + k * NS

        @pl.when(b < CHUNKS)
        def _():
            pltpu.async_copy(msg[3], acc.at[pl.ds(b * C, C)], zsem)

        return 0

    lax.fori_loop(0, -(-CHUNKS // NS), zero_blk, 0)

    def zero_drain(k, _):
        b = sid + k * NS

        @pl.when(b < CHUNKS)
        def _():
            pltpu.make_async_copy(emb_hbm.at[pl.ds(0, C)], msg[3],
                                  zsem).wait()

        return 0

    lax.fori_loop(0, -(-CHUNKS // NS), zero_drain, 0)
    plsc.subcore_barrier()

    def quad(k, _):
        for j in range(NB):
            step(NB * k + j, j)
        return 0

    lax.fori_loop(0, (CHUNKS - 1) // NB, quad, 0)
    wait_scatter(2)
    wait_gather(0)
    scale(0)
    wait_edr(0)
    start_scatter(CHUNKS - 1, 0)
    wait_scatter(3)
    wait_scatter(0)
    plsc.subcore_barrier()

    def out_body(k, _):
        b = sid + k * NS

        @pl.when(b < NOB)
        def _():
            row0 = b * OBLK
            pltpu.sync_copy(acc.at[pl.ds(row0, OBLK)],
                            part_hbm.at[cid, pl.ds(row0, OBLK)])

        return 0

    lax.fori_loop(0, -(-NOB // NS), out_body, 0)


def _combine_body(p_ref, e_ref, o_ref):
    o_ref[...] = p_ref[0] + p_ref[1] + 0.25 * e_ref[...]


def kernel(g_indices, g_values, emb_weight):
    rows = g_indices[0].astype(jnp.int32).reshape(NW, CHUNKS, 1, C)
    cols = g_indices[1].astype(jnp.int32).reshape(NW, CHUNKS, C)
    gvb = jax.lax.bitcast_convert_type(
        0.75 * g_values.astype(jnp.float32), jnp.int32).reshape(NW, CHUNKS, C)
    edc = jnp.stack([cols, gvb], axis=2)
    emb = emb_weight.astype(jnp.float32)

    mesh = plsc.VectorSubcoreMesh(core_axis_name="c", subcore_axis_name="s")
    spmm = functools.partial(
        pl.kernel,
        out_type=jax.ShapeDtypeStruct((NC, N, D), jnp.float32),
        mesh=mesh,
        scratch_types=[
            pltpu.VMEM_SHARED((N, D), jnp.float32),
            [pltpu.VMEM((C, D), jnp.float32) for _ in range(NB)],
            [pltpu.VMEM((1, C), jnp.int32) for _ in range(NB)],
            [pltpu.VMEM((2, C), jnp.int32) for _ in range(NB)],
            [pltpu.SemaphoreType.DMA for _ in range(NB)],
            [pltpu.SemaphoreType.DMA for _ in range(NB)],
            [pltpu.SemaphoreType.DMA for _ in range(NB)],
            [pltpu.SemaphoreType.DMA for _ in range(NB)],
            pltpu.SemaphoreType.DMA,
        ],
    )(_sc_spmm)
    partials = spmm(rows, edc, emb)

    blk = 1000
    out = pl.pallas_call(
        _combine_body,
        grid=(N // blk,),
        in_specs=[
            pl.BlockSpec((NC, blk, D), lambda i: (0, i, 0)),
            pl.BlockSpec((blk, D), lambda i: (i, 0)),
        ],
        out_specs=pl.BlockSpec((blk, D), lambda i: (i, 0)),
        out_shape=jax.ShapeDtypeStruct((N, D), jnp.float32),
    )(partials, emb)
    return out
